# Initial kernel scaffold; baseline (speedup 1.0000x reference)
import jax
import jax.numpy as jnp
from jax.experimental import pallas as pl


def kernel(x):
    raise NotImplementedError("write your pallas kernel here")


# speedup vs baseline: 40.2962x; 40.2962x over previous
import dataclasses
import math
import jax
import jax.numpy as jnp
from jax import lax
from jax.experimental import pallas as pl
from jax.experimental.pallas import tpu as pltpu
from jax.experimental.pallas import tpu_sc as plsc

_K_KDE = 15
_K_RIPS = 15
_SCALE = 2.0
_DESTNUM = 5
_N = 4096
_L = 16


def _pair_d2(x):
    sq = jnp.sum(x * x, axis=1)
    d2 = sq[:, None] + sq[None, :] - 2.0 * (x @ x.T)
    return jnp.maximum(d2, 0.0)


def _knn(x, k):
    d2 = _pair_d2(x)
    neg_vals, idxs = jax.lax.top_k(-d2, k + 1)
    return -neg_vals[:, 1:], idxs[:, 1:]


def _cluster_sc(dens_sorted, rips_padded):
    n = dens_sorted.shape[0]
    mesh = plsc.VectorSubcoreMesh(core_axis_name="c", subcore_axis_name="s")
    cp = pltpu.CompilerParams()
    if "needs_layout_passes" in pltpu.CompilerParams.__dataclass_fields__:
        cp = dataclasses.replace(cp, needs_layout_passes=False)

    @pl.kernel(
        compiler_params=cp,
        out_type=(
            jax.ShapeDtypeStruct((n,), jnp.int32),
            jax.ShapeDtypeStruct((n,), jnp.int32),
        ),
        mesh=mesh,
        scratch_types=[
            pltpu.VMEM((n,), jnp.float32),
            pltpu.VMEM((n * _L,), jnp.int32),
            pltpu.VMEM((n,), jnp.int32),
            pltpu.VMEM((n,), jnp.int32),
            pltpu.VMEM((n,), jnp.int32),
        ],
    )
    def k(dens_hbm, rips_hbm, pres_out, sec_out,
          dens_v, rips_v, root_v, sec_v, pres_v):
        ci = lax.axis_index("c")
        si = lax.axis_index("s")

        @pl.when((ci == 0) & (si == 0))
        def _():
            pltpu.sync_copy(dens_hbm, dens_v)
            pltpu.sync_copy(rips_hbm, rips_v)
            lanes = lax.iota(jnp.int32, _L)

            @pl.loop(0, n // _L)
            def _(b):
                base = b * _L
                sl = pl.ds(base, _L)
                root_v[sl] = lanes + base
                sec_v[sl] = jnp.zeros((_L,), jnp.int32)
                pres_v[sl] = jnp.zeros((_L,), jnp.int32)

            @pl.loop(0, n)
            def _(t):
                i = (n - 1) - t
                nb = rips_v[pl.ds(i * _L, _L)]
                vm = nb > i
                has = jnp.any(vm)

                @pl.when(has)
                def _():
                    j0 = jnp.where(vm, nb, i)

                    def cond(c):
                        return c[1]

                    def body(c):
                        j, _ = c
                        p = plsc.load_gather(root_v, [j])
                        return p, jnp.any(p != j)

                    r, _unused = lax.while_loop(
                        cond, body, (j0, jnp.bool_(True)))
                    dens_r = plsc.load_gather(dens_v, [r])
                    dmax = jnp.max(jnp.where(vm, dens_r, -jnp.inf))
                    surv = jnp.max(
                        jnp.where(vm & (dens_r == dmax), r, -1))
                    g = jnp.max(jnp.where(vm, nb, -1))
                    ri0 = jnp.max(jnp.where(nb == g, r, -1))
                    lane0 = lanes == 0
                    plsc.store_scatter(
                        root_v, [jnp.full((_L,), i, jnp.int32)],
                        jnp.full((_L,), ri0, jnp.int32), mask=lane0)
                    kill = vm & (r != surv)
                    plsc.store_scatter(
                        root_v, [r], jnp.full((_L,), surv, jnp.int32),
                        mask=kill)
                    plsc.store_scatter(
                        sec_v, [r], jnp.full((_L,), i, jnp.int32),
                        mask=kill)
                    plsc.store_scatter(
                        pres_v, [r], jnp.ones((_L,), jnp.int32),
                        mask=kill)

            @pl.loop(0, n // _L)
            def _(b):
                base = b * _L
                sl = pl.ds(base, _L)
                rv = root_v[sl]
                idv = lanes + base
                isr = rv == idv
                sec_v[sl] = jnp.where(isr, 0, sec_v[sl])
                pres_v[sl] = jnp.where(isr, 1, pres_v[sl])

            pltpu.sync_copy(pres_v, pres_out)
            pltpu.sync_copy(sec_v, sec_out)

    return k(dens_sorted, rips_padded)


def kernel(x):
    n = x.shape[0]
    knn_d2, _ = _knn(x, _K_KDE)
    dens = jnp.sum(jnp.exp(-knn_d2 / _SCALE), axis=1) / (_K_KDE * _SCALE)
    sorted_idxs = jnp.argsort(dens)
    dens_sorted = dens[sorted_idxs]
    xs = x[sorted_idxs]
    rips_idxs = _knn(xs, _K_RIPS)[1]

    iota = jnp.arange(n, dtype=jnp.int32)
    rips_padded = jnp.concatenate(
        [rips_idxs.astype(jnp.int32), iota[:, None]], axis=1).reshape(-1)
    present_i, second_safe = _cluster_sc(dens_sorted, rips_padded)
    present = present_i > 0

    pers = dens_sorted - dens_sorted[second_safe]
    sidx = jnp.argsort(jnp.where(present, pers, jnp.inf))
    m = jnp.sum(present.astype(jnp.int32))
    vb = sidx[m - 1]
    dest = jnp.stack(
        [dens_sorted[vb], dens_sorted[second_safe[vb]]]).astype(jnp.float32)
    pairs_sorted = jnp.stack([sidx, second_safe[sidx]], axis=1)
    idxv = iota
    changemask = idxv < m - _DESTNUM
    nochangemask = (idxv >= m - _DESTNUM) & (idxv <= m - 2)
    pd11 = dens_sorted[pairs_sorted]
    diffs = pd11[:, 0] - pd11[:, 1]
    weakdist = jnp.sum(jnp.where(changemask, diffs, 0.0)) / math.sqrt(2)
    norms = jnp.linalg.norm(pd11 - dest[None, :], axis=1)
    strongdist = jnp.sum(jnp.where(nochangemask, norms, 0.0))
    loss = weakdist + strongdist
    return loss, rips_idxs


# speedup vs baseline: 263.0771x; 6.5286x over previous
import dataclasses
import math
import jax
import jax.numpy as jnp
from jax import lax
from jax.experimental import pallas as pl
from jax.experimental.pallas import tpu as pltpu
from jax.experimental.pallas import tpu_sc as plsc

_K_KDE = 15
_K_RIPS = 15
_SCALE = 2.0
_DESTNUM = 5
_N = 4096
_L = 16


def _pair_d2(x):
    sq = jnp.sum(x * x, axis=1)
    d2 = sq[:, None] + sq[None, :] - 2.0 * (x @ x.T)
    return jnp.maximum(d2, 0.0)


def _knn(x, k):
    d2 = _pair_d2(x)
    neg_vals, idxs = jax.lax.top_k(-d2, k + 1)
    return -neg_vals[:, 1:], idxs[:, 1:]


_BLK = 256


def _knn16_tc(x):
    n, d = x.shape
    sq2 = jnp.sum(x * x, axis=1)[None, :]

    def body(xb_ref, sqb_ref, xall_ref, sq_ref, vals_ref, idxs_ref):
        xall = xall_ref[...]
        xb = xb_ref[...]
        sq = sq_ref[0, :]
        sqb = sqb_ref[0, :]
        dot = jax.lax.dot_general(
            xb, xall, (((1,), (1,)), ((), ())),
            preferred_element_type=jnp.float32)
        d2 = sqb[:, None] + sq[None, :] - 2.0 * dot
        work = jnp.maximum(d2, 0.0)
        iota = jax.lax.broadcasted_iota(jnp.int32, (_BLK, n), 1)
        vals, idxs = [], []
        for _ in range(16):
            m = jnp.min(work, axis=1, keepdims=True)
            hit = work == m
            ix = jnp.min(jnp.where(hit, iota, n), axis=1, keepdims=True)
            vals.append(m)
            idxs.append(ix)
            work = jnp.where(iota == ix, jnp.inf, work)
        vals_ref[...] = jnp.concatenate(vals, axis=1)
        idxs_ref[...] = jnp.concatenate(idxs, axis=1)

    vals, idxs = pl.pallas_call(
        body,
        grid=(n // _BLK,),
        in_specs=[
            pl.BlockSpec((_BLK, d), lambda i: (i, 0)),
            pl.BlockSpec((1, _BLK), lambda i: (0, i)),
            pl.BlockSpec((n, d), lambda i: (0, 0)),
            pl.BlockSpec((1, n), lambda i: (0, 0)),
        ],
        out_specs=[
            pl.BlockSpec((_BLK, 16), lambda i: (i, 0)),
            pl.BlockSpec((_BLK, 16), lambda i: (i, 0)),
        ],
        out_shape=[
            jax.ShapeDtypeStruct((n, 16), jnp.float32),
            jax.ShapeDtypeStruct((n, 16), jnp.int32),
        ],
    )(x, sq2, x, sq2)
    return vals, idxs


def _cluster_sc(dens_sorted, rips_padded):
    n = dens_sorted.shape[0]
    mesh = plsc.VectorSubcoreMesh(core_axis_name="c", subcore_axis_name="s")
    cp = pltpu.CompilerParams()
    if "needs_layout_passes" in pltpu.CompilerParams.__dataclass_fields__:
        cp = dataclasses.replace(cp, needs_layout_passes=False)

    @pl.kernel(
        compiler_params=cp,
        out_type=(
            jax.ShapeDtypeStruct((n,), jnp.int32),
            jax.ShapeDtypeStruct((n,), jnp.int32),
        ),
        mesh=mesh,
        scratch_types=[
            pltpu.VMEM((n,), jnp.float32),
            pltpu.VMEM((n * _L,), jnp.int32),
            pltpu.VMEM((n,), jnp.int32),
            pltpu.VMEM((n,), jnp.int32),
            pltpu.VMEM((n,), jnp.int32),
        ],
    )
    def k(dens_hbm, rips_hbm, pres_out, sec_out,
          dens_v, rips_v, root_v, sec_v, pres_v):
        ci = lax.axis_index("c")
        si = lax.axis_index("s")

        @pl.when((ci == 0) & (si == 0))
        def _():
            pltpu.sync_copy(dens_hbm, dens_v)
            pltpu.sync_copy(rips_hbm, rips_v)
            lanes = lax.iota(jnp.int32, _L)

            @pl.loop(0, n // _L)
            def _(b):
                base = b * _L
                sl = pl.ds(base, _L)
                root_v[sl] = lanes + base
                sec_v[sl] = jnp.zeros((_L,), jnp.int32)
                pres_v[sl] = jnp.zeros((_L,), jnp.int32)

            @pl.loop(0, n)
            def _(t):
                i = (n - 1) - t
                nb = rips_v[pl.ds(i * _L, _L)]
                vm = nb > i
                has = jnp.any(vm)

                @pl.when(has)
                def _():
                    j0 = jnp.where(vm, nb, i)

                    def cond(c):
                        return c[1]

                    def body(c):
                        j, _ = c
                        p = plsc.load_gather(root_v, [j])
                        return p, jnp.any(p != j)

                    r, _unused = lax.while_loop(
                        cond, body, (j0, jnp.bool_(True)))
                    dens_r = plsc.load_gather(dens_v, [r])
                    dmax = jnp.max(jnp.where(vm, dens_r, -jnp.inf))
                    surv = jnp.max(
                        jnp.where(vm & (dens_r == dmax), r, -1))
                    g = jnp.max(jnp.where(vm, nb, -1))
                    ri0 = jnp.max(jnp.where(nb == g, r, -1))
                    lane0 = lanes == 0
                    plsc.store_scatter(
                        root_v, [jnp.full((_L,), i, jnp.int32)],
                        jnp.full((_L,), ri0, jnp.int32), mask=lane0)
                    kill = vm & (r != surv)
                    plsc.store_scatter(
                        root_v, [r], jnp.full((_L,), surv, jnp.int32),
                        mask=kill)
                    plsc.store_scatter(
                        sec_v, [r], jnp.full((_L,), i, jnp.int32),
                        mask=kill)
                    plsc.store_scatter(
                        pres_v, [r], jnp.ones((_L,), jnp.int32),
                        mask=kill)

            @pl.loop(0, n // _L)
            def _(b):
                base = b * _L
                sl = pl.ds(base, _L)
                rv = root_v[sl]
                idv = lanes + base
                isr = rv == idv
                sec_v[sl] = jnp.where(isr, 0, sec_v[sl])
                pres_v[sl] = jnp.where(isr, 1, pres_v[sl])

            pltpu.sync_copy(pres_v, pres_out)
            pltpu.sync_copy(sec_v, sec_out)

    return k(dens_sorted, rips_padded)


def kernel(x):
    n = x.shape[0]
    knn_d2 = _knn16_tc(x)[0][:, 1:]
    dens = jnp.sum(jnp.exp(-knn_d2 / _SCALE), axis=1) / (_K_KDE * _SCALE)
    sorted_idxs = jnp.argsort(dens)
    dens_sorted = dens[sorted_idxs]
    xs = x[sorted_idxs]
    rips_idxs = _knn16_tc(xs)[1][:, 1:]

    iota = jnp.arange(n, dtype=jnp.int32)
    rips_padded = jnp.concatenate(
        [rips_idxs.astype(jnp.int32), iota[:, None]], axis=1).reshape(-1)
    present_i, second_safe = _cluster_sc(dens_sorted, rips_padded)
    present = present_i > 0

    pers = dens_sorted - dens_sorted[second_safe]
    sidx = jnp.argsort(jnp.where(present, pers, jnp.inf))
    m = jnp.sum(present.astype(jnp.int32))
    vb = sidx[m - 1]
    dest = jnp.stack(
        [dens_sorted[vb], dens_sorted[second_safe[vb]]]).astype(jnp.float32)
    pairs_sorted = jnp.stack([sidx, second_safe[sidx]], axis=1)
    idxv = iota
    changemask = idxv < m - _DESTNUM
    nochangemask = (idxv >= m - _DESTNUM) & (idxv <= m - 2)
    pd11 = dens_sorted[pairs_sorted]
    diffs = pd11[:, 0] - pd11[:, 1]
    weakdist = jnp.sum(jnp.where(changemask, diffs, 0.0)) / math.sqrt(2)
    norms = jnp.linalg.norm(pd11 - dest[None, :], axis=1)
    strongdist = jnp.sum(jnp.where(nochangemask, norms, 0.0))
    loss = weakdist + strongdist
    return loss, rips_idxs


# speedup vs baseline: 288.3569x; 1.0961x over previous
import dataclasses
import math
import jax
import jax.numpy as jnp
from jax import lax
from jax.experimental import pallas as pl
from jax.experimental.pallas import tpu as pltpu
from jax.experimental.pallas import tpu_sc as plsc

_K_KDE = 15
_K_RIPS = 15
_SCALE = 2.0
_DESTNUM = 5
_N = 4096
_L = 16


def _pair_d2(x):
    sq = jnp.sum(x * x, axis=1)
    d2 = sq[:, None] + sq[None, :] - 2.0 * (x @ x.T)
    return jnp.maximum(d2, 0.0)


def _knn(x, k):
    d2 = _pair_d2(x)
    neg_vals, idxs = jax.lax.top_k(-d2, k + 1)
    return -neg_vals[:, 1:], idxs[:, 1:]


_BLK = 256


def _knn16_tc(x):
    n, d = x.shape
    sq2 = jnp.sum(x * x, axis=1)[None, :]

    def body(xb_ref, sqb_ref, xall_ref, sq_ref, vals_ref, idxs_ref):
        xall = xall_ref[...]
        xb = xb_ref[...]
        sq = sq_ref[0, :]
        sqb = sqb_ref[0, :]
        dot = jax.lax.dot_general(
            xb, xall, (((1,), (1,)), ((), ())),
            preferred_element_type=jnp.float32)
        d2 = sqb[:, None] + sq[None, :] - 2.0 * dot
        work = jnp.maximum(d2, 0.0)
        iota = jax.lax.broadcasted_iota(jnp.int32, (_BLK, n), 1)
        vals, idxs = [], []
        for _ in range(16):
            m = jnp.min(work, axis=1, keepdims=True)
            hit = work == m
            ix = jnp.min(jnp.where(hit, iota, n), axis=1, keepdims=True)
            vals.append(m)
            idxs.append(ix)
            work = jnp.where(iota == ix, jnp.inf, work)
        vals_ref[...] = jnp.concatenate(vals, axis=1)
        idxs_ref[...] = jnp.concatenate(idxs, axis=1)

    vals, idxs = pl.pallas_call(
        body,
        grid=(n // _BLK,),
        in_specs=[
            pl.BlockSpec((_BLK, d), lambda i: (i, 0)),
            pl.BlockSpec((1, _BLK), lambda i: (0, i)),
            pl.BlockSpec((n, d), lambda i: (0, 0)),
            pl.BlockSpec((1, n), lambda i: (0, 0)),
        ],
        out_specs=[
            pl.BlockSpec((_BLK, 16), lambda i: (i, 0)),
            pl.BlockSpec((_BLK, 16), lambda i: (i, 0)),
        ],
        out_shape=[
            jax.ShapeDtypeStruct((n, 16), jnp.float32),
            jax.ShapeDtypeStruct((n, 16), jnp.int32),
        ],
    )(x, sq2, x, sq2)
    return vals, idxs


def _cluster_sc(dens_sorted, rips_padded):
    n = dens_sorted.shape[0]
    mesh = plsc.VectorSubcoreMesh(core_axis_name="c", subcore_axis_name="s")
    cp = pltpu.CompilerParams()
    if "needs_layout_passes" in pltpu.CompilerParams.__dataclass_fields__:
        cp = dataclasses.replace(cp, needs_layout_passes=False)

    @pl.kernel(
        compiler_params=cp,
        out_type=jax.ShapeDtypeStruct((_L,), jnp.float32),
        mesh=mesh,
        scratch_types=[
            pltpu.VMEM((n,), jnp.float32),
            pltpu.VMEM((n * _L,), jnp.int32),
            pltpu.VMEM((n,), jnp.int32),
            pltpu.VMEM((n,), jnp.int32),
            pltpu.VMEM((n,), jnp.int32),
            pltpu.VMEM((n,), jnp.float32),
            pltpu.VMEM((n,), jnp.int32),
            pltpu.VMEM((_L,), jnp.float32),
        ],
    )
    def k(dens_hbm, rips_hbm, stats_out,
          dens_v, rips_v, root_v, sec_v, pres_v, pers_v, act_v, out_v):
        ci = lax.axis_index("c")
        si = lax.axis_index("s")

        @pl.when((ci == 0) & (si == 0))
        def _():
            pltpu.sync_copy(dens_hbm, dens_v)
            pltpu.sync_copy(rips_hbm, rips_v)
            lanes = lax.iota(jnp.int32, _L)

            @pl.loop(0, n // _L)
            def _(b):
                base = b * _L
                sl = pl.ds(base, _L)
                root_v[sl] = lanes + base
                sec_v[sl] = jnp.zeros((_L,), jnp.int32)
                pres_v[sl] = jnp.zeros((_L,), jnp.int32)

            @pl.loop(0, n)
            def _(t):
                i = (n - 1) - t
                nb = rips_v[pl.ds(i * _L, _L)]
                vm = nb > i
                has = jnp.any(vm)

                @pl.when(has)
                def _():
                    j0 = jnp.where(vm, nb, i)

                    def cond(c):
                        return c[1]

                    def body(c):
                        j, _ = c
                        p = plsc.load_gather(root_v, [j])
                        return p, jnp.any(p != j)

                    r, _unused = lax.while_loop(
                        cond, body, (j0, jnp.bool_(True)))
                    plsc.store_scatter(root_v, [nb], r, mask=vm)
                    dens_r = plsc.load_gather(dens_v, [r])
                    dmax = jnp.max(jnp.where(vm, dens_r, -jnp.inf))
                    surv = jnp.max(
                        jnp.where(vm & (dens_r == dmax), r, -1))
                    g = jnp.max(jnp.where(vm, nb, -1))
                    ri0 = jnp.max(jnp.where(nb == g, r, -1))
                    lane0 = lanes == 0
                    plsc.store_scatter(
                        root_v, [jnp.full((_L,), i, jnp.int32)],
                        jnp.full((_L,), ri0, jnp.int32), mask=lane0)
                    kill = vm & (r != surv)
                    plsc.store_scatter(
                        root_v, [r], jnp.full((_L,), surv, jnp.int32),
                        mask=kill)
                    plsc.store_scatter(
                        sec_v, [r], jnp.full((_L,), i, jnp.int32),
                        mask=kill)
                    plsc.store_scatter(
                        pres_v, [r], jnp.ones((_L,), jnp.int32),
                        mask=kill)

            def sweep(b, carry):
                acc, mcnt = carry
                base = b * _L
                sl = pl.ds(base, _L)
                rv = root_v[sl]
                idv = lanes + base
                isr = rv == idv
                sec = jnp.where(isr, 0, sec_v[sl])
                pres = jnp.where(isr, 1, pres_v[sl])
                dgat = plsc.load_gather(dens_v, [sec])
                pers = dens_v[sl] - dgat
                pers_v[sl] = pers
                act_v[sl] = pres
                acc = acc + jnp.where(pres > 0, pers, 0.0)
                mcnt = mcnt + pres
                return acc, mcnt

            acc, mcnt = lax.fori_loop(
                0, n // _L, sweep,
                (jnp.zeros((_L,), jnp.float32),
                 jnp.zeros((_L,), jnp.int32)))
            total = jnp.sum(acc)
            m = jnp.sum(mcnt)

            out = jnp.where(lanes == 0, total, 0.0)
            out = out + jnp.where(
                lanes == 11, m.astype(jnp.float32), 0.0)

            def pick(p, out):
                def vpass(b, vmax):
                    sl = pl.ds(b * _L, _L)
                    a = act_v[sl] > 0
                    return jnp.maximum(
                        vmax, jnp.where(a, pers_v[sl], -jnp.inf))

                pmax = jnp.max(lax.fori_loop(
                    0, n // _L, vpass,
                    jnp.full((_L,), -jnp.inf, jnp.float32)))

                def ipass(b, vbest):
                    base = b * _L
                    sl = pl.ds(base, _L)
                    a = (act_v[sl] > 0) & (pers_v[sl] == pmax)
                    return jnp.maximum(
                        vbest, jnp.where(a, lanes + base, -1))

                e = jnp.max(lax.fori_loop(
                    0, n // _L, ipass,
                    jnp.full((_L,), -1, jnp.int32)))
                valid = e >= 0
                e_safe = jnp.maximum(e, 0)
                e_vec = jnp.full((_L,), e_safe, jnp.int32)
                dval = jnp.max(plsc.load_gather(dens_v, [e_vec]))
                plsc.store_scatter(
                    act_v, [e_vec], jnp.zeros((_L,), jnp.int32),
                    mask=(lanes == 0) & valid)
                pval = jnp.where(valid, pmax, 0.0)
                dval = jnp.where(valid, dval, 0.0)
                out = out + jnp.where(lanes == 1 + p, pval, 0.0)
                out = out + jnp.where(lanes == 6 + p, dval, 0.0)
                return out

            out = lax.fori_loop(0, 5, pick, out)
            out_v[...] = out
            pltpu.sync_copy(out_v, stats_out)

    return k(dens_sorted, rips_padded)


def kernel(x):
    n = x.shape[0]
    knn_d2 = _knn16_tc(x)[0][:, 1:]
    dens = jnp.sum(jnp.exp(-knn_d2 / _SCALE), axis=1) / (_K_KDE * _SCALE)
    sorted_idxs = jnp.argsort(dens)
    dens_sorted = dens[sorted_idxs]
    xs = x[sorted_idxs]
    rips_idxs = _knn16_tc(xs)[1][:, 1:]

    iota = jnp.arange(n, dtype=jnp.int32)
    rips_padded = jnp.concatenate(
        [rips_idxs.astype(jnp.int32), iota[:, None]], axis=1).reshape(-1)
    stats = _cluster_sc(dens_sorted, rips_padded)

    total = stats[0]
    pvals = stats[1:6]
    dvals = stats[6:11]
    m = stats[11]
    pidx = jnp.arange(1, 6, dtype=jnp.float32)
    pvalid = pidx <= m
    weakdist = (total - jnp.sum(jnp.where(pvalid, pvals, 0.0))) / math.sqrt(2)
    dest0 = dvals[0]
    dest1 = dvals[0] - pvals[0]
    nrm = jnp.sqrt((dvals - dest0) ** 2 + (dvals - pvals - dest1) ** 2)
    strongdist = jnp.sum(jnp.where(pvalid & (pidx >= 2), nrm, 0.0))
    loss = weakdist + strongdist
    return loss, rips_idxs


# speedup vs baseline: 349.7582x; 1.2129x over previous
import dataclasses
import math
import jax
import jax.numpy as jnp
from jax import lax
from jax.experimental import pallas as pl
from jax.experimental.pallas import tpu as pltpu
from jax.experimental.pallas import tpu_sc as plsc

_K_KDE = 15
_K_RIPS = 15
_SCALE = 2.0
_DESTNUM = 5
_N = 4096
_L = 16


def _pair_d2(x):
    sq = jnp.sum(x * x, axis=1)
    d2 = sq[:, None] + sq[None, :] - 2.0 * (x @ x.T)
    return jnp.maximum(d2, 0.0)


def _knn(x, k):
    d2 = _pair_d2(x)
    neg_vals, idxs = jax.lax.top_k(-d2, k + 1)
    return -neg_vals[:, 1:], idxs[:, 1:]


_BLK = 256


def _knn16_tc(x):
    n, d = x.shape
    sq2 = jnp.sum(x * x, axis=1)[None, :]

    def body(xb_ref, sqb_ref, xall_ref, sq_ref, vals_ref, idxs_ref):
        xall = xall_ref[...]
        xb = xb_ref[...]
        sq = sq_ref[0, :]
        sqb = sqb_ref[0, :]
        dot = jax.lax.dot_general(
            xb, xall, (((1,), (1,)), ((), ())),
            preferred_element_type=jnp.float32)
        d2 = sqb[:, None] + sq[None, :] - 2.0 * dot
        work = jnp.maximum(d2, 0.0)
        iota = jax.lax.broadcasted_iota(jnp.int32, (_BLK, n), 1)
        vals, idxs = [], []
        for _ in range(16):
            m = jnp.min(work, axis=1, keepdims=True)
            hit = work == m
            ix = jnp.min(jnp.where(hit, iota, n), axis=1, keepdims=True)
            vals.append(m)
            idxs.append(ix)
            work = jnp.where(iota == ix, jnp.inf, work)
        vals_ref[...] = jnp.concatenate(vals, axis=1)
        idxs_ref[...] = jnp.concatenate(idxs, axis=1)

    vals, idxs = pl.pallas_call(
        body,
        grid=(n // _BLK,),
        in_specs=[
            pl.BlockSpec((_BLK, d), lambda i: (i, 0)),
            pl.BlockSpec((1, _BLK), lambda i: (0, i)),
            pl.BlockSpec((n, d), lambda i: (0, 0)),
            pl.BlockSpec((1, n), lambda i: (0, 0)),
        ],
        out_specs=[
            pl.BlockSpec((_BLK, 16), lambda i: (i, 0)),
            pl.BlockSpec((_BLK, 16), lambda i: (i, 0)),
        ],
        out_shape=[
            jax.ShapeDtypeStruct((n, 16), jnp.float32),
            jax.ShapeDtypeStruct((n, 16), jnp.int32),
        ],
    )(x, sq2, x, sq2)
    return vals, idxs


def _cluster_sc(dens_sorted, rips_padded):
    n = dens_sorted.shape[0]
    mesh = plsc.VectorSubcoreMesh(core_axis_name="c", subcore_axis_name="s")
    cp = pltpu.CompilerParams()
    if "needs_layout_passes" in pltpu.CompilerParams.__dataclass_fields__:
        cp = dataclasses.replace(cp, needs_layout_passes=False)

    @pl.kernel(
        compiler_params=cp,
        out_type=jax.ShapeDtypeStruct((_L,), jnp.float32),
        mesh=mesh,
        scratch_types=[
            pltpu.VMEM((n,), jnp.float32),
            pltpu.VMEM((n * _L,), jnp.int32),
            pltpu.VMEM((n,), jnp.int32),
            pltpu.VMEM((n,), jnp.int32),
            pltpu.VMEM((n,), jnp.float32),
            pltpu.VMEM((n,), jnp.int32),
            pltpu.VMEM((_L,), jnp.float32),
        ],
    )
    def k(dens_hbm, rips_hbm, stats_out,
          dens_v, rips_v, root_v, sec_v, pers_v, act_v, out_v):
        ci = lax.axis_index("c")
        si = lax.axis_index("s")

        @pl.when((ci == 0) & (si == 0))
        def _():
            pltpu.sync_copy(dens_hbm, dens_v)
            pltpu.sync_copy(rips_hbm, rips_v)
            lanes = lax.iota(jnp.int32, _L)

            @pl.loop(0, n // _L)
            def _(b):
                base = b * _L
                sl = pl.ds(base, _L)
                root_v[sl] = lanes + base
                sec_v[sl] = jnp.full((_L,), -1, jnp.int32)

            lane15 = lanes == 15

            @pl.loop(0, n)
            def _(t):
                i = (n - 1) - t
                nb = rips_v[pl.ds(i * _L, _L)]
                vm = nb > i
                j0 = jnp.where(vm, nb, i)
                r1 = plsc.load_gather(root_v, [j0])
                r2 = plsc.load_gather(root_v, [r1])
                conv = jnp.all(r2 == r1)

                def fix(r):
                    def cond(c):
                        return c[1]

                    def body(c):
                        j, _ = c
                        p = plsc.load_gather(root_v, [j])
                        return p, jnp.any(p != j)

                    return lax.while_loop(
                        cond, body, (r, jnp.bool_(True)))[0]

                r = lax.cond(conv, lambda rr: rr, fix, r2)
                plsc.store_scatter(root_v, [nb], r, mask=vm)
                dens_r = plsc.load_gather(dens_v, [r])
                dmax = jnp.max(jnp.where(vm, dens_r, -jnp.inf))
                surv = jnp.max(jnp.where(vm & (dens_r == dmax), r, -1))
                pk = jnp.max(
                    jnp.where(vm, (nb << 12) | r, -1))
                ri0 = jnp.where(pk >= 0, pk & 0xFFF, i)
                kill = vm & (r != surv)
                idxs = jnp.where(lane15, i, r)
                vals = jnp.where(lane15, ri0, surv)
                plsc.store_scatter(
                    root_v, [idxs], vals, mask=kill | lane15)
                plsc.store_scatter(
                    sec_v, [r], jnp.full((_L,), i, jnp.int32),
                    mask=kill)

            def sweep(b, carry):
                acc, mcnt = carry
                base = b * _L
                sl = pl.ds(base, _L)
                rv = root_v[sl]
                idv = lanes + base
                isr = rv == idv
                secraw = sec_v[sl]
                pres = jnp.where(isr | (secraw >= 0), 1, 0)
                sec = jnp.where(isr, 0, jnp.maximum(secraw, 0))
                dgat = plsc.load_gather(dens_v, [sec])
                pers = dens_v[sl] - dgat
                pers_v[sl] = pers
                act_v[sl] = pres
                acc = acc + jnp.where(pres > 0, pers, 0.0)
                mcnt = mcnt + pres
                return acc, mcnt

            acc, mcnt = lax.fori_loop(
                0, n // _L, sweep,
                (jnp.zeros((_L,), jnp.float32),
                 jnp.zeros((_L,), jnp.int32)))
            total = jnp.sum(acc)
            m = jnp.sum(mcnt)

            out = jnp.where(lanes == 0, total, 0.0)
            out = out + jnp.where(
                lanes == 11, m.astype(jnp.float32), 0.0)

            def pick(p, out):
                def vpass(b, vmax):
                    sl = pl.ds(b * _L, _L)
                    a = act_v[sl] > 0
                    return jnp.maximum(
                        vmax, jnp.where(a, pers_v[sl], -jnp.inf))

                pmax = jnp.max(lax.fori_loop(
                    0, n // _L, vpass,
                    jnp.full((_L,), -jnp.inf, jnp.float32)))

                def ipass(b, vbest):
                    base = b * _L
                    sl = pl.ds(base, _L)
                    a = (act_v[sl] > 0) & (pers_v[sl] == pmax)
                    return jnp.maximum(
                        vbest, jnp.where(a, lanes + base, -1))

                e = jnp.max(lax.fori_loop(
                    0, n // _L, ipass,
                    jnp.full((_L,), -1, jnp.int32)))
                valid = e >= 0
                e_safe = jnp.maximum(e, 0)
                e_vec = jnp.full((_L,), e_safe, jnp.int32)
                dval = jnp.max(plsc.load_gather(dens_v, [e_vec]))
                plsc.store_scatter(
                    act_v, [e_vec], jnp.zeros((_L,), jnp.int32),
                    mask=(lanes == 0) & valid)
                pval = jnp.where(valid, pmax, 0.0)
                dval = jnp.where(valid, dval, 0.0)
                out = out + jnp.where(lanes == 1 + p, pval, 0.0)
                out = out + jnp.where(lanes == 6 + p, dval, 0.0)
                return out

            out = lax.fori_loop(0, 5, pick, out)
            out_v[...] = out
            pltpu.sync_copy(out_v, stats_out)

    return k(dens_sorted, rips_padded)


def kernel(x):
    n = x.shape[0]
    knn_d2 = _knn16_tc(x)[0][:, 1:]
    dens = jnp.sum(jnp.exp(-knn_d2 / _SCALE), axis=1) / (_K_KDE * _SCALE)
    sorted_idxs = jnp.argsort(dens)
    dens_sorted = dens[sorted_idxs]
    xs = x[sorted_idxs]
    rips_idxs = _knn16_tc(xs)[1][:, 1:]

    iota = jnp.arange(n, dtype=jnp.int32)
    rips_padded = jnp.concatenate(
        [rips_idxs.astype(jnp.int32), iota[:, None]], axis=1).reshape(-1)
    stats = _cluster_sc(dens_sorted, rips_padded)

    total = stats[0]
    pvals = stats[1:6]
    dvals = stats[6:11]
    m = stats[11]
    pidx = jnp.arange(1, 6, dtype=jnp.float32)
    pvalid = pidx <= m
    weakdist = (total - jnp.sum(jnp.where(pvalid, pvals, 0.0))) / math.sqrt(2)
    dest0 = dvals[0]
    dest1 = dvals[0] - pvals[0]
    nrm = jnp.sqrt((dvals - dest0) ** 2 + (dvals - pvals - dest1) ** 2)
    strongdist = jnp.sum(jnp.where(pvalid & (pidx >= 2), nrm, 0.0))
    loss = weakdist + strongdist
    return loss, rips_idxs
